# baseline (speedup 1.0000x reference)
import jax
import jax.numpy as jnp
from jax import lax
from jax.experimental import pallas as pl
from jax.experimental.pallas import tpu as pltpu
from jax.experimental.pallas import tpu_sc as plsc

N_NODES = 10000
N_EDGES = 320000
D = 128

NC = 2
NS = 16

CHUNK = 100
NCHUNKS = N_EDGES // CHUNK
ROWS_PER_TILE = NCHUNKS // NS

NODE_BLK = 632
NODE_BLK_LAST = N_NODES - NODE_BLK * (NS - 1)
ZBLK = 96


def _zero_acc_range(acc, rows_v, base, count):
    nfull = count // ZBLK
    tail = count - nfull * ZBLK
    for k in range(nfull):
        pltpu.sync_copy(rows_v.at[pl.ds(0, ZBLK)],
                        acc.at[pl.ds(base + k * ZBLK, ZBLK)])
    if tail:
        pltpu.sync_copy(rows_v.at[pl.ds(0, tail)],
                        acc.at[pl.ds(base + nfull * ZBLK, tail)])


def _segsum_body(dest_hbm, attr_hbm, out_hbm, acc, idx_v, rows_v):
    s = lax.axis_index("s")

    def zero_row(r, _):
        for k in range(D // 16):
            rows_v[r, pl.ds(k * 16, 16)] = jnp.zeros((16,), jnp.float32)
        return 0
    lax.fori_loop(0, CHUNK, zero_row, 0)

    @pl.when(s < NS - 1)
    def _():
        _zero_acc_range(acc, rows_v, s * NODE_BLK, NODE_BLK)

    @pl.when(s == NS - 1)
    def _():
        _zero_acc_range(acc, rows_v, (NS - 1) * NODE_BLK, NODE_BLK_LAST)

    plsc.subcore_barrier()

    rbase = s * ROWS_PER_TILE

    pltpu.sync_copy(dest_hbm.at[pl.ds(rbase, ROWS_PER_TILE)], idx_v)

    def chunk_body(j, _):
        pltpu.sync_copy(attr_hbm.at[rbase + j], rows_v)
        pltpu.sync_copy(rows_v, acc.at[idx_v.at[j]], add=True)
        return 0
    lax.fori_loop(0, ROWS_PER_TILE, chunk_body, 0)

    plsc.subcore_barrier()

    @pl.when(s < NS - 1)
    def _():
        pltpu.sync_copy(acc.at[pl.ds(s * NODE_BLK, NODE_BLK)],
                        out_hbm.at[pl.ds(s * NODE_BLK, NODE_BLK)])

    @pl.when(s == NS - 1)
    def _():
        pltpu.sync_copy(acc.at[pl.ds((NS - 1) * NODE_BLK, NODE_BLK_LAST)],
                        out_hbm.at[pl.ds((NS - 1) * NODE_BLK, NODE_BLK_LAST)])


def _segment_one(dest, attr):
    mesh = plsc.VectorSubcoreMesh(core_axis_name="c", subcore_axis_name="s",
                                  num_cores=1, num_subcores=NS)
    f = pl.kernel(
        _segsum_body,
        out_type=jax.ShapeDtypeStruct((N_NODES, D), jnp.float32),
        mesh=mesh,
        scratch_types=[
            pltpu.VMEM_SHARED((N_NODES, D), jnp.float32),
            pltpu.VMEM((ROWS_PER_TILE, CHUNK), jnp.int32),
            pltpu.VMEM((CHUNK, D), jnp.float32),
        ],
    )
    return f(dest, attr)


def _segment_sums(destw, destm, attrw, attrm):
    return _segment_one(destw, attrw), _segment_one(destm, attrm)


def _mlp_body(x_ref, ow_ref, om_ref, w0x_ref, w0w_ref, w0m_ref, b0_ref,
              w1_ref, b1_ref, w2_ref, b2_ref, out_ref):
    def silu(h):
        return h * (1.0 / (1.0 + jnp.exp(-h)))
    h = (jnp.dot(x_ref[...], w0x_ref[...], preferred_element_type=jnp.float32)
         + jnp.dot(ow_ref[...], w0w_ref[...], preferred_element_type=jnp.float32)
         + jnp.dot(om_ref[...], w0m_ref[...], preferred_element_type=jnp.float32)
         + b0_ref[...])
    h = silu(h)
    h = silu(jnp.dot(h, w1_ref[...], preferred_element_type=jnp.float32) + b1_ref[...])
    out_ref[...] = (jnp.dot(h, w2_ref[...], preferred_element_type=jnp.float32)
                    + b2_ref[...])


def _mlp(x, outw, outm, W0, b0, W1, b1, W2, b2):
    blk = 1000
    grid = (N_NODES // blk,)
    row_spec = pl.BlockSpec((blk, D), lambda i: (i, 0))
    w_spec = pl.BlockSpec((D, D), lambda i: (0, 0))
    b_spec = pl.BlockSpec((1, D), lambda i: (0, 0))
    return pl.pallas_call(
        _mlp_body,
        grid=grid,
        in_specs=[row_spec, row_spec, row_spec,
                  w_spec, w_spec, w_spec, b_spec,
                  w_spec, b_spec, w_spec, b_spec],
        out_specs=row_spec,
        out_shape=jax.ShapeDtypeStruct((N_NODES, D), jnp.float32),
    )(x, outw, outm, W0[0:D], W0[D:2 * D], W0[2 * D:3 * D], b0.reshape(1, D),
      W1, b1.reshape(1, D), W2, b2.reshape(1, D))


def kernel(x, edge_indexw, edge_indexm, edge_attrw, edge_attrm,
           W0, b0, W1, b1, W2, b2):
    destw = edge_indexw[1].astype(jnp.int32).reshape(NCHUNKS, CHUNK)
    destm = edge_indexm[1].astype(jnp.int32).reshape(NCHUNKS, CHUNK)
    attrw = edge_attrw.reshape(NCHUNKS, CHUNK, D)
    attrm = edge_attrm.reshape(NCHUNKS, CHUNK, D)
    outw, outm = _segment_sums(destw, destm, attrw, attrm)
    return _mlp(x, outw, outm, W0, b0, W1, b1, W2, b2)


# speedup vs baseline: 1.4744x; 1.4744x over previous
import jax
import jax.numpy as jnp
from jax import lax
from jax.experimental import pallas as pl
from jax.experimental.pallas import tpu as pltpu
from jax.experimental.pallas import tpu_sc as plsc

N_NODES = 10000
N_EDGES = 320000
D = 128

NC = 2
NS = 16

CHUNK = 100
NCHUNKS = N_EDGES // CHUNK
ROWS_PER_TILE = NCHUNKS // NS
NB = 2
SEG0 = 104

NODE_BLK = 632
NODE_BLK_LAST = N_NODES - NODE_BLK * (NS - 1)
ZBLK = 96


def _zero_acc_range(acc, rows_v, base, count):
    nfull = count // ZBLK
    tail = count - nfull * ZBLK
    for k in range(nfull):
        pltpu.sync_copy(rows_v.at[0, pl.ds(0, ZBLK)],
                        acc.at[pl.ds(base + k * ZBLK, ZBLK)])
    if tail:
        pltpu.sync_copy(rows_v.at[0, pl.ds(0, tail)],
                        acc.at[pl.ds(base + nfull * ZBLK, tail)])


def _segsum_body(destw_hbm, destm_hbm, attrw_hbm, attrm_hbm,
                 outw_hbm, outm_hbm, acc, idx_v, rows_v,
                 lsem0, lsem1, ssem0, ssem1):
    c = lax.axis_index("c")
    s = lax.axis_index("s")
    lsems = (lsem0, lsem1)
    ssems = (ssem0, ssem1)

    def zero_row(r, _):
        for k in range(D // 16):
            rows_v[0, r, pl.ds(k * 16, 16)] = jnp.zeros((16,), jnp.float32)
        return 0
    lax.fori_loop(0, ZBLK, zero_row, 0)

    @pl.when(s < NS - 1)
    def _():
        _zero_acc_range(acc, rows_v, s * NODE_BLK, NODE_BLK)

    @pl.when(s == NS - 1)
    def _():
        _zero_acc_range(acc, rows_v, (NS - 1) * NODE_BLK, NODE_BLK_LAST)

    plsc.subcore_barrier()

    rbase = s * ROWS_PER_TILE

    def scatter_edges(dest_hbm, attr_hbm):
        for seg_off, seg_len in ((0, SEG0), (SEG0, ROWS_PER_TILE - SEG0)):
            pltpu.sync_copy(dest_hbm.at[pl.ds(rbase + seg_off, seg_len)],
                            idx_v.at[pl.ds(0, seg_len)])

            for b in range(NB - 1):
                pltpu.async_copy(attr_hbm.at[rbase + seg_off + b],
                                 rows_v.at[b], lsems[b])

            def chunk_body(j, _):
                def step(b):
                    pltpu.make_async_copy(attr_hbm.at[rbase + seg_off + j],
                                          rows_v.at[b], lsems[b]).wait()
                    pltpu.async_copy(rows_v.at[b], acc.at[idx_v.at[j]],
                                     ssems[b], add=True)
                    bn = (b + NB - 1) % NB
                    jn = j + NB - 1

                    @pl.when(jn < seg_len)
                    def _():
                        @pl.when(j >= 1)
                        def _():
                            pltpu.make_async_copy(rows_v.at[bn],
                                                  acc.at[idx_v.at[j - 1]],
                                                  ssems[bn]).wait()
                        pltpu.async_copy(attr_hbm.at[rbase + seg_off + jn],
                                         rows_v.at[bn], lsems[bn])

                for b in range(NB):
                    @pl.when(j % NB == b)
                    def _(b=b):
                        step(b)
                return 0

            lax.fori_loop(0, seg_len, chunk_body, 0)

            for k in range(NB):
                j = seg_len - NB + k
                pltpu.make_async_copy(rows_v.at[j % NB], acc.at[idx_v.at[j]],
                                      ssems[j % NB]).wait()

    @pl.when(c == 0)
    def _():
        scatter_edges(destw_hbm, attrw_hbm)

    @pl.when(c == 1)
    def _():
        scatter_edges(destm_hbm, attrm_hbm)

    plsc.subcore_barrier()

    def writeout(out_hbm):
        @pl.when(s < NS - 1)
        def _():
            pltpu.sync_copy(acc.at[pl.ds(s * NODE_BLK, NODE_BLK)],
                            out_hbm.at[pl.ds(s * NODE_BLK, NODE_BLK)])

        @pl.when(s == NS - 1)
        def _():
            pltpu.sync_copy(acc.at[pl.ds((NS - 1) * NODE_BLK, NODE_BLK_LAST)],
                            out_hbm.at[pl.ds((NS - 1) * NODE_BLK, NODE_BLK_LAST)])

    @pl.when(c == 0)
    def _():
        writeout(outw_hbm)

    @pl.when(c == 1)
    def _():
        writeout(outm_hbm)


def _segment_sums(destw, destm, attrw, attrm):
    mesh = plsc.VectorSubcoreMesh(core_axis_name="c", subcore_axis_name="s",
                                  num_cores=NC, num_subcores=NS)
    f = pl.kernel(
        _segsum_body,
        out_type=(jax.ShapeDtypeStruct((N_NODES, D), jnp.float32),
                  jax.ShapeDtypeStruct((N_NODES, D), jnp.float32)),
        mesh=mesh,
        scratch_types=[
            pltpu.VMEM_SHARED((N_NODES, D), jnp.float32),
            pltpu.VMEM((SEG0, CHUNK), jnp.int32),
            pltpu.VMEM((NB, CHUNK, D), jnp.float32),
        ] + [pltpu.SemaphoreType.DMA] * (2 * NB),
    )
    return f(destw, destm, attrw, attrm)


def _mlp_body(x_ref, ow_ref, om_ref, w0x_ref, w0w_ref, w0m_ref, b0_ref,
              w1_ref, b1_ref, w2_ref, b2_ref, out_ref):
    def silu(h):
        return h * (1.0 / (1.0 + jnp.exp(-h)))
    h = (jnp.dot(x_ref[...], w0x_ref[...], preferred_element_type=jnp.float32)
         + jnp.dot(ow_ref[...], w0w_ref[...], preferred_element_type=jnp.float32)
         + jnp.dot(om_ref[...], w0m_ref[...], preferred_element_type=jnp.float32)
         + b0_ref[...])
    h = silu(h)
    h = silu(jnp.dot(h, w1_ref[...], preferred_element_type=jnp.float32) + b1_ref[...])
    out_ref[...] = (jnp.dot(h, w2_ref[...], preferred_element_type=jnp.float32)
                    + b2_ref[...])


def _mlp(x, outw, outm, W0, b0, W1, b1, W2, b2):
    blk = 1000
    grid = (N_NODES // blk,)
    row_spec = pl.BlockSpec((blk, D), lambda i: (i, 0))
    w_spec = pl.BlockSpec((D, D), lambda i: (0, 0))
    b_spec = pl.BlockSpec((1, D), lambda i: (0, 0))
    return pl.pallas_call(
        _mlp_body,
        grid=grid,
        in_specs=[row_spec, row_spec, row_spec,
                  w_spec, w_spec, w_spec, b_spec,
                  w_spec, b_spec, w_spec, b_spec],
        out_specs=row_spec,
        out_shape=jax.ShapeDtypeStruct((N_NODES, D), jnp.float32),
    )(x, outw, outm, W0[0:D], W0[D:2 * D], W0[2 * D:3 * D], b0.reshape(1, D),
      W1, b1.reshape(1, D), W2, b2.reshape(1, D))


def kernel(x, edge_indexw, edge_indexm, edge_attrw, edge_attrm,
           W0, b0, W1, b1, W2, b2):
    destw = edge_indexw[1].astype(jnp.int32).reshape(NCHUNKS, CHUNK)
    destm = edge_indexm[1].astype(jnp.int32).reshape(NCHUNKS, CHUNK)
    attrw = edge_attrw.reshape(NCHUNKS, CHUNK, D)
    attrm = edge_attrm.reshape(NCHUNKS, CHUNK, D)
    outw, outm = _segment_sums(destw, destm, attrw, attrm)
    return _mlp(x, outw, outm, W0, b0, W1, b1, W2, b2)


# speedup vs baseline: 3.1623x; 2.1448x over previous
import jax
import jax.numpy as jnp
from jax import lax
from jax.experimental import pallas as pl
from jax.experimental.pallas import tpu as pltpu
from jax.experimental.pallas import tpu_sc as plsc

N_NODES = 10000
N_EDGES = 320000
D = 128

NC = 2
NS = 16

CHUNK = 128
NCHUNKS = N_EDGES // CHUNK
RPT = 160
RPT_LAST = NCHUNKS - RPT * (NS - 1)
NCHUNKS_PAD = RPT * NS
NB = 2
SEG = 80

NODE_BLK = 632
NODE_BLK_LAST = N_NODES - NODE_BLK * (NS - 1)
ZBLK = 96


def _zero_acc_range(acc, rows_v, base, count):
    nfull = count // ZBLK
    tail = count - nfull * ZBLK
    for k in range(nfull):
        pltpu.sync_copy(rows_v.at[0, pl.ds(0, ZBLK)],
                        acc.at[pl.ds(base + k * ZBLK, ZBLK)])
    if tail:
        pltpu.sync_copy(rows_v.at[0, pl.ds(0, tail)],
                        acc.at[pl.ds(base + nfull * ZBLK, tail)])


def _segsum_body(destw_hbm, destm_hbm, attrw_hbm, attrm_hbm,
                 outw_hbm, outm_hbm, acc, idx_v, rows_v,
                 lsem0, lsem1, ssem0, ssem1):
    c = lax.axis_index("c")
    s = lax.axis_index("s")
    lsems = (lsem0, lsem1)
    ssems = (ssem0, ssem1)

    def zero_row(r, _):
        for k in range(D // 16):
            rows_v[0, r, pl.ds(k * 16, 16)] = jnp.zeros((16,), jnp.float32)
        return 0
    lax.fori_loop(0, ZBLK, zero_row, 0)

    @pl.when(s < NS - 1)
    def _():
        _zero_acc_range(acc, rows_v, s * NODE_BLK, NODE_BLK)

    @pl.when(s == NS - 1)
    def _():
        _zero_acc_range(acc, rows_v, (NS - 1) * NODE_BLK, NODE_BLK_LAST)

    plsc.subcore_barrier()

    rbase = s * RPT
    nrows = jnp.where(s == NS - 1, RPT_LAST, RPT)

    def scatter_edges(dest_hbm, attr_hbm):
        for g in range(RPT // SEG):
            seg_off = g * SEG
            seg_len = jnp.clip(nrows - seg_off, 0, SEG)

            @pl.when(seg_len > 0)
            def _():
                pltpu.sync_copy(dest_hbm.at[pl.ds(rbase + seg_off, SEG)],
                                idx_v)

                for b in range(NB - 1):
                    pltpu.async_copy(attr_hbm.at[rbase + seg_off + b],
                                     rows_v.at[b], lsems[b])

                def chunk_body(j, _):
                    def step(b):
                        pltpu.make_async_copy(attr_hbm.at[rbase + seg_off + j],
                                              rows_v.at[b], lsems[b]).wait()
                        pltpu.async_copy(rows_v.at[b], acc.at[idx_v.at[j]],
                                         ssems[b], add=True)
                        bn = (b + NB - 1) % NB
                        jn = j + NB - 1

                        @pl.when(jn < seg_len)
                        def _():
                            @pl.when(j >= 1)
                            def _():
                                pltpu.make_async_copy(rows_v.at[bn],
                                                      acc.at[idx_v.at[j - 1]],
                                                      ssems[bn]).wait()
                            pltpu.async_copy(attr_hbm.at[rbase + seg_off + jn],
                                             rows_v.at[bn], lsems[bn])

                    for b in range(NB):
                        @pl.when(j % NB == b)
                        def _(b=b):
                            step(b)
                    return 0

                lax.fori_loop(0, seg_len, chunk_body, 0)

                for k in range(NB):
                    pltpu.make_async_copy(rows_v.at[k],
                                          acc.at[idx_v.at[seg_len - NB + k]],
                                          ssems[k]).wait()

    @pl.when(c == 0)
    def _():
        scatter_edges(destw_hbm, attrw_hbm)

    @pl.when(c == 1)
    def _():
        scatter_edges(destm_hbm, attrm_hbm)

    plsc.subcore_barrier()

    def writeout(out_hbm):
        @pl.when(s < NS - 1)
        def _():
            pltpu.sync_copy(acc.at[pl.ds(s * NODE_BLK, NODE_BLK)],
                            out_hbm.at[pl.ds(s * NODE_BLK, NODE_BLK)])

        @pl.when(s == NS - 1)
        def _():
            pltpu.sync_copy(acc.at[pl.ds((NS - 1) * NODE_BLK, NODE_BLK_LAST)],
                            out_hbm.at[pl.ds((NS - 1) * NODE_BLK, NODE_BLK_LAST)])

    @pl.when(c == 0)
    def _():
        writeout(outw_hbm)

    @pl.when(c == 1)
    def _():
        writeout(outm_hbm)


def _segment_sums(destw, destm, attrw, attrm):
    mesh = plsc.VectorSubcoreMesh(core_axis_name="c", subcore_axis_name="s",
                                  num_cores=NC, num_subcores=NS)
    f = pl.kernel(
        _segsum_body,
        out_type=(jax.ShapeDtypeStruct((N_NODES, D), jnp.float32),
                  jax.ShapeDtypeStruct((N_NODES, D), jnp.float32)),
        mesh=mesh,
        scratch_types=[
            pltpu.VMEM_SHARED((N_NODES, D), jnp.float32),
            pltpu.VMEM((SEG, CHUNK), jnp.int32),
            pltpu.VMEM((NB, CHUNK, D), jnp.float32),
        ] + [pltpu.SemaphoreType.DMA] * (2 * NB),
    )
    return f(destw, destm, attrw, attrm)


def _mlp_body(x_ref, ow_ref, om_ref, w0x_ref, w0w_ref, w0m_ref, b0_ref,
              w1_ref, b1_ref, w2_ref, b2_ref, out_ref):
    def silu(h):
        return h * (1.0 / (1.0 + jnp.exp(-h)))
    h = (jnp.dot(x_ref[...], w0x_ref[...], preferred_element_type=jnp.float32)
         + jnp.dot(ow_ref[...], w0w_ref[...], preferred_element_type=jnp.float32)
         + jnp.dot(om_ref[...], w0m_ref[...], preferred_element_type=jnp.float32)
         + b0_ref[...])
    h = silu(h)
    h = silu(jnp.dot(h, w1_ref[...], preferred_element_type=jnp.float32) + b1_ref[...])
    out_ref[...] = (jnp.dot(h, w2_ref[...], preferred_element_type=jnp.float32)
                    + b2_ref[...])


def _mlp(x, outw, outm, W0, b0, W1, b1, W2, b2):
    blk = 1000
    grid = (N_NODES // blk,)
    row_spec = pl.BlockSpec((blk, D), lambda i: (i, 0))
    w_spec = pl.BlockSpec((D, D), lambda i: (0, 0))
    b_spec = pl.BlockSpec((1, D), lambda i: (0, 0))
    return pl.pallas_call(
        _mlp_body,
        grid=grid,
        in_specs=[row_spec, row_spec, row_spec,
                  w_spec, w_spec, w_spec, b_spec,
                  w_spec, b_spec, w_spec, b_spec],
        out_specs=row_spec,
        out_shape=jax.ShapeDtypeStruct((N_NODES, D), jnp.float32),
    )(x, outw, outm, W0[0:D], W0[D:2 * D], W0[2 * D:3 * D], b0.reshape(1, D),
      W1, b1.reshape(1, D), W2, b2.reshape(1, D))


def _dest_2d(edge_index):
    d = edge_index[1].astype(jnp.int32)
    pad = NCHUNKS_PAD * CHUNK - N_EDGES
    d = jnp.concatenate([d, jnp.zeros((pad,), jnp.int32)])
    return d.reshape(NCHUNKS_PAD, CHUNK)


def kernel(x, edge_indexw, edge_indexm, edge_attrw, edge_attrm,
           W0, b0, W1, b1, W2, b2):
    destw = _dest_2d(edge_indexw)
    destm = _dest_2d(edge_indexm)
    attrw = edge_attrw.reshape(NCHUNKS, CHUNK, D)
    attrm = edge_attrm.reshape(NCHUNKS, CHUNK, D)
    outw, outm = _segment_sums(destw, destm, attrw, attrm)
    return _mlp(x, outw, outm, W0, b0, W1, b1, W2, b2)


# speedup vs baseline: 3.7235x; 1.1775x over previous
import jax
import jax.numpy as jnp
from jax import lax
from jax.experimental import pallas as pl
from jax.experimental.pallas import tpu as pltpu
from jax.experimental.pallas import tpu_sc as plsc

N_NODES = 10000
N_EDGES = 320000
D = 128

NC = 2
NS = 16

CHUNK = 64
NCHUNKS = N_EDGES // CHUNK
RPT = 312
RPT_LAST = NCHUNKS - RPT * (NS - 1)
NB = 4
SEG = 80
NSEG = (RPT_LAST + SEG - 1) // SEG

NODE_BLK = 632
NODE_BLK_LAST = N_NODES - NODE_BLK * (NS - 1)
ZBLK = 64


def _zero_acc_range(acc, rows_v, base, count):
    nfull = count // ZBLK
    tail = count - nfull * ZBLK
    for k in range(nfull):
        pltpu.sync_copy(rows_v.at[0],
                        acc.at[pl.ds(base + k * ZBLK, ZBLK)])
    if tail:
        pltpu.sync_copy(rows_v.at[0, pl.ds(0, tail)],
                        acc.at[pl.ds(base + nfull * ZBLK, tail)])


def _segsum_body(destw_hbm, destm_hbm, attrw_hbm, attrm_hbm,
                 outw_hbm, outm_hbm, acc, idx_v, rows_v,
                 lsem0, lsem1, lsem2, lsem3, ssem0, ssem1, ssem2, ssem3):
    c = lax.axis_index("c")
    s = lax.axis_index("s")
    lsems = (lsem0, lsem1, lsem2, lsem3)
    ssems = (ssem0, ssem1, ssem2, ssem3)

    def zero_row(r, _):
        for k in range(D // 16):
            rows_v[0, r, pl.ds(k * 16, 16)] = jnp.zeros((16,), jnp.float32)
        return 0
    lax.fori_loop(0, ZBLK, zero_row, 0)

    @pl.when(s < NS - 1)
    def _():
        _zero_acc_range(acc, rows_v, s * NODE_BLK, NODE_BLK)

    @pl.when(s == NS - 1)
    def _():
        _zero_acc_range(acc, rows_v, (NS - 1) * NODE_BLK, NODE_BLK_LAST)

    plsc.subcore_barrier()

    rbase = s * RPT
    nrows = jnp.where(s == NS - 1, RPT_LAST, RPT)

    def scatter_edges(dest_hbm, attr_hbm):
        for g in range(NSEG):
            seg_off = g * SEG
            seg_len = jnp.clip(nrows - seg_off, 0, SEG)

            @pl.when(seg_len > 0)
            def _():
                pltpu.sync_copy(dest_hbm.at[pl.ds(rbase + seg_off, SEG)],
                                idx_v)

                for b in range(NB - 1):
                    pltpu.async_copy(attr_hbm.at[rbase + seg_off + b],
                                     rows_v.at[b], lsems[b])

                def chunk_body(j, _):
                    def step(b):
                        pltpu.make_async_copy(attr_hbm.at[rbase + seg_off + j],
                                              rows_v.at[b], lsems[b]).wait()
                        pltpu.async_copy(rows_v.at[b], acc.at[idx_v.at[j]],
                                         ssems[b], add=True)
                        bn = (b + NB - 1) % NB
                        jn = j + NB - 1

                        @pl.when(jn < seg_len)
                        def _():
                            @pl.when(j >= 1)
                            def _():
                                pltpu.make_async_copy(rows_v.at[bn],
                                                      acc.at[idx_v.at[j - 1]],
                                                      ssems[bn]).wait()
                            pltpu.async_copy(attr_hbm.at[rbase + seg_off + jn],
                                             rows_v.at[bn], lsems[bn])

                    for b in range(NB):
                        @pl.when(j % NB == b)
                        def _(b=b):
                            step(b)
                    return 0

                lax.fori_loop(0, seg_len, chunk_body, 0)

                for k in range(NB):
                    pltpu.make_async_copy(rows_v.at[k],
                                          acc.at[idx_v.at[seg_len - NB + k]],
                                          ssems[k]).wait()

    @pl.when(c == 0)
    def _():
        scatter_edges(destw_hbm, attrw_hbm)

    @pl.when(c == 1)
    def _():
        scatter_edges(destm_hbm, attrm_hbm)

    plsc.subcore_barrier()

    def writeout(out_hbm):
        @pl.when(s < NS - 1)
        def _():
            pltpu.sync_copy(acc.at[pl.ds(s * NODE_BLK, NODE_BLK)],
                            out_hbm.at[pl.ds(s * NODE_BLK, NODE_BLK)])

        @pl.when(s == NS - 1)
        def _():
            pltpu.sync_copy(acc.at[pl.ds((NS - 1) * NODE_BLK, NODE_BLK_LAST)],
                            out_hbm.at[pl.ds((NS - 1) * NODE_BLK, NODE_BLK_LAST)])

    @pl.when(c == 0)
    def _():
        writeout(outw_hbm)

    @pl.when(c == 1)
    def _():
        writeout(outm_hbm)


def _segment_sums(destw, destm, attrw, attrm):
    mesh = plsc.VectorSubcoreMesh(core_axis_name="c", subcore_axis_name="s",
                                  num_cores=NC, num_subcores=NS)
    f = pl.kernel(
        _segsum_body,
        out_type=(jax.ShapeDtypeStruct((N_NODES, D), jnp.float32),
                  jax.ShapeDtypeStruct((N_NODES, D), jnp.float32)),
        mesh=mesh,
        scratch_types=[
            pltpu.VMEM_SHARED((N_NODES, D), jnp.float32),
            pltpu.VMEM((SEG, CHUNK), jnp.int32),
            pltpu.VMEM((NB, CHUNK, D), jnp.float32),
        ] + [pltpu.SemaphoreType.DMA] * (2 * NB),
    )
    return f(destw, destm, attrw, attrm)


def _mlp_body(x_ref, ow_ref, om_ref, w0x_ref, w0w_ref, w0m_ref, b0_ref,
              w1_ref, b1_ref, w2_ref, b2_ref, out_ref):
    def silu(h):
        return h * (1.0 / (1.0 + jnp.exp(-h)))
    h = (jnp.dot(x_ref[...], w0x_ref[...], preferred_element_type=jnp.float32)
         + jnp.dot(ow_ref[...], w0w_ref[...], preferred_element_type=jnp.float32)
         + jnp.dot(om_ref[...], w0m_ref[...], preferred_element_type=jnp.float32)
         + b0_ref[...])
    h = silu(h)
    h = silu(jnp.dot(h, w1_ref[...], preferred_element_type=jnp.float32) + b1_ref[...])
    out_ref[...] = (jnp.dot(h, w2_ref[...], preferred_element_type=jnp.float32)
                    + b2_ref[...])


def _mlp(x, outw, outm, W0, b0, W1, b1, W2, b2):
    blk = 1000
    grid = (N_NODES // blk,)
    row_spec = pl.BlockSpec((blk, D), lambda i: (i, 0))
    w_spec = pl.BlockSpec((D, D), lambda i: (0, 0))
    b_spec = pl.BlockSpec((1, D), lambda i: (0, 0))
    return pl.pallas_call(
        _mlp_body,
        grid=grid,
        in_specs=[row_spec, row_spec, row_spec,
                  w_spec, w_spec, w_spec, b_spec,
                  w_spec, b_spec, w_spec, b_spec],
        out_specs=row_spec,
        out_shape=jax.ShapeDtypeStruct((N_NODES, D), jnp.float32),
    )(x, outw, outm, W0[0:D], W0[D:2 * D], W0[2 * D:3 * D], b0.reshape(1, D),
      W1, b1.reshape(1, D), W2, b2.reshape(1, D))


def kernel(x, edge_indexw, edge_indexm, edge_attrw, edge_attrm,
           W0, b0, W1, b1, W2, b2):
    destw = edge_indexw[1].astype(jnp.int32).reshape(NCHUNKS, CHUNK)
    destm = edge_indexm[1].astype(jnp.int32).reshape(NCHUNKS, CHUNK)
    attrw = edge_attrw.reshape(NCHUNKS, CHUNK, D)
    attrm = edge_attrm.reshape(NCHUNKS, CHUNK, D)
    outw, outm = _segment_sums(destw, destm, attrw, attrm)
    return _mlp(x, outw, outm, W0, b0, W1, b1, W2, b2)


# speedup vs baseline: 4.0087x; 1.0766x over previous
import jax
import jax.numpy as jnp
from jax import lax
from jax.experimental import pallas as pl
from jax.experimental.pallas import tpu as pltpu
from jax.experimental.pallas import tpu_sc as plsc

N_NODES = 10000
N_EDGES = 320000
D = 128

NC = 2
NS = 16

CHUNK = 128
NCHUNKS = N_EDGES // CHUNK
RPT = NCHUNKS // NS
REM = NCHUNKS - RPT * NS
NB = 3

NODE_BLK = 632
NODE_BLK_LAST = N_NODES - NODE_BLK * (NS - 1)
ZBLK = 128


def _zero_acc_range(acc, rows_v, base, count):
    nfull = count // ZBLK
    tail = count - nfull * ZBLK
    for k in range(nfull):
        pltpu.sync_copy(rows_v.at[0],
                        acc.at[pl.ds(base + k * ZBLK, ZBLK)])
    if tail:
        pltpu.sync_copy(rows_v.at[0, pl.ds(0, tail)],
                        acc.at[pl.ds(base + nfull * ZBLK, tail)])


def _segsum_body(destw_hbm, destm_hbm, attrw_hbm, attrm_hbm,
                 outw_hbm, outm_hbm, acc, idx_r, rows_v,
                 lsem0, lsem1, lsem2, ssem0, ssem1, ssem2):
    c = lax.axis_index("c")
    s = lax.axis_index("s")
    lsems = (lsem0, lsem1, lsem2)
    ssems = (ssem0, ssem1, ssem2)

    def zero_row(r, _):
        for k in range(D // 16):
            rows_v[0, r, pl.ds(k * 16, 16)] = jnp.zeros((16,), jnp.float32)
        return 0
    lax.fori_loop(0, ZBLK, zero_row, 0)

    @pl.when(s < NS - 1)
    def _():
        _zero_acc_range(acc, rows_v, s * NODE_BLK, NODE_BLK)

    @pl.when(s == NS - 1)
    def _():
        _zero_acc_range(acc, rows_v, (NS - 1) * NODE_BLK, NODE_BLK_LAST)

    plsc.subcore_barrier()

    rbase = s * RPT + jnp.minimum(s, REM)
    nrows = jnp.where(s < REM, RPT + 1, RPT)

    def scatter_edges(dest_hbm, attr_hbm):
        def fire_load(b, jj):
            pltpu.async_copy(dest_hbm.at[pl.ds((rbase + jj) * CHUNK, CHUNK)],
                             idx_r.at[b], lsems[b])
            pltpu.async_copy(attr_hbm.at[rbase + jj], rows_v.at[b], lsems[b])

        def wait_load(b, jj):
            pltpu.make_async_copy(dest_hbm.at[pl.ds((rbase + jj) * CHUNK,
                                                    CHUNK)],
                                  idx_r.at[b], lsems[b]).wait()
            pltpu.make_async_copy(attr_hbm.at[rbase + jj],
                                  rows_v.at[b], lsems[b]).wait()

        for b in range(NB - 1):
            fire_load(b, b)

        def chunk_body(j, _):
            def step(b):
                wait_load(b, j)
                pltpu.async_copy(rows_v.at[b], acc.at[idx_r.at[b]],
                                 ssems[b], add=True)
                bn = (b + NB - 1) % NB
                jn = j + NB - 1

                @pl.when(jn < nrows)
                def _():
                    @pl.when(j >= 1)
                    def _():
                        pltpu.make_async_copy(rows_v.at[bn],
                                              acc.at[idx_r.at[bn]],
                                              ssems[bn]).wait()
                    fire_load(bn, jn)

            for b in range(NB):
                @pl.when(j % NB == b)
                def _(b=b):
                    step(b)
            return 0

        lax.fori_loop(0, nrows, chunk_body, 0)

        for b in range(NB):
            pltpu.make_async_copy(rows_v.at[b], acc.at[idx_r.at[b]],
                                  ssems[b]).wait()

    @pl.when(c == 0)
    def _():
        scatter_edges(destw_hbm, attrw_hbm)

    @pl.when(c == 1)
    def _():
        scatter_edges(destm_hbm, attrm_hbm)

    plsc.subcore_barrier()

    def writeout(out_hbm):
        @pl.when(s < NS - 1)
        def _():
            pltpu.sync_copy(acc.at[pl.ds(s * NODE_BLK, NODE_BLK)],
                            out_hbm.at[pl.ds(s * NODE_BLK, NODE_BLK)])

        @pl.when(s == NS - 1)
        def _():
            pltpu.sync_copy(acc.at[pl.ds((NS - 1) * NODE_BLK, NODE_BLK_LAST)],
                            out_hbm.at[pl.ds((NS - 1) * NODE_BLK, NODE_BLK_LAST)])

    @pl.when(c == 0)
    def _():
        writeout(outw_hbm)

    @pl.when(c == 1)
    def _():
        writeout(outm_hbm)


def _segment_sums(destw, destm, attrw, attrm):
    mesh = plsc.VectorSubcoreMesh(core_axis_name="c", subcore_axis_name="s",
                                  num_cores=NC, num_subcores=NS)
    f = pl.kernel(
        _segsum_body,
        out_type=(jax.ShapeDtypeStruct((N_NODES, D), jnp.float32),
                  jax.ShapeDtypeStruct((N_NODES, D), jnp.float32)),
        mesh=mesh,
        scratch_types=[
            pltpu.VMEM_SHARED((N_NODES, D), jnp.float32),
            pltpu.VMEM((NB, CHUNK), jnp.int32),
            pltpu.VMEM((NB, CHUNK, D), jnp.float32),
        ] + [pltpu.SemaphoreType.DMA] * (2 * NB),
    )
    return f(destw, destm, attrw, attrm)


def _mlp_body(x_ref, ow_ref, om_ref, w0x_ref, w0w_ref, w0m_ref, b0_ref,
              w1_ref, b1_ref, w2_ref, b2_ref, out_ref):
    def silu(h):
        return h * (1.0 / (1.0 + jnp.exp(-h)))
    h = (jnp.dot(x_ref[...], w0x_ref[...], preferred_element_type=jnp.float32)
         + jnp.dot(ow_ref[...], w0w_ref[...], preferred_element_type=jnp.float32)
         + jnp.dot(om_ref[...], w0m_ref[...], preferred_element_type=jnp.float32)
         + b0_ref[...])
    h = silu(h)
    h = silu(jnp.dot(h, w1_ref[...], preferred_element_type=jnp.float32) + b1_ref[...])
    out_ref[...] = (jnp.dot(h, w2_ref[...], preferred_element_type=jnp.float32)
                    + b2_ref[...])


def _mlp(x, outw, outm, W0, b0, W1, b1, W2, b2):
    blk = 1000
    grid = (N_NODES // blk,)
    row_spec = pl.BlockSpec((blk, D), lambda i: (i, 0))
    w_spec = pl.BlockSpec((D, D), lambda i: (0, 0))
    b_spec = pl.BlockSpec((1, D), lambda i: (0, 0))
    return pl.pallas_call(
        _mlp_body,
        grid=grid,
        in_specs=[row_spec, row_spec, row_spec,
                  w_spec, w_spec, w_spec, b_spec,
                  w_spec, b_spec, w_spec, b_spec],
        out_specs=row_spec,
        out_shape=jax.ShapeDtypeStruct((N_NODES, D), jnp.float32),
    )(x, outw, outm, W0[0:D], W0[D:2 * D], W0[2 * D:3 * D], b0.reshape(1, D),
      W1, b1.reshape(1, D), W2, b2.reshape(1, D))


def kernel(x, edge_indexw, edge_indexm, edge_attrw, edge_attrm,
           W0, b0, W1, b1, W2, b2):
    destw = edge_indexw[1].astype(jnp.int32)
    destm = edge_indexm[1].astype(jnp.int32)
    attrw = edge_attrw.reshape(NCHUNKS, CHUNK, D)
    attrm = edge_attrm.reshape(NCHUNKS, CHUNK, D)
    outw, outm = _segment_sums(destw, destm, attrw, attrm)
    return _mlp(x, outw, outm, W0, b0, W1, b1, W2, b2)


# speedup vs baseline: 4.1103x; 1.0253x over previous
import jax
import jax.numpy as jnp
from jax import lax
from jax.experimental import pallas as pl
from jax.experimental.pallas import tpu as pltpu
from jax.experimental.pallas import tpu_sc as plsc

N_NODES = 10000
N_EDGES = 320000
D = 128

NC = 2
NS = 16

CHUNK = 80
NCHUNKS = N_EDGES // CHUNK
RPT = NCHUNKS // NS
REM = NCHUNKS - RPT * NS
NB = 4

NODE_BLK = 632
NODE_BLK_LAST = N_NODES - NODE_BLK * (NS - 1)
ZBLK = 80


def _zero_acc_range(acc, rows_v, base, count):
    nfull = count // ZBLK
    tail = count - nfull * ZBLK
    for k in range(nfull):
        pltpu.sync_copy(rows_v.at[0],
                        acc.at[pl.ds(base + k * ZBLK, ZBLK)])
    if tail:
        pltpu.sync_copy(rows_v.at[0, pl.ds(0, tail)],
                        acc.at[pl.ds(base + nfull * ZBLK, tail)])


def _segsum_body(destw_hbm, destm_hbm, attrw_hbm, attrm_hbm,
                 outw_hbm, outm_hbm, acc, idx_r, rows_v,
                 lsem0, lsem1, lsem2, lsem3, ssem0, ssem1, ssem2, ssem3):
    c = lax.axis_index("c")
    s = lax.axis_index("s")
    lsems = (lsem0, lsem1, lsem2, lsem3)
    ssems = (ssem0, ssem1, ssem2, ssem3)

    def zero_row(r, _):
        for k in range(D // 16):
            rows_v[0, r, pl.ds(k * 16, 16)] = jnp.zeros((16,), jnp.float32)
        return 0
    lax.fori_loop(0, ZBLK, zero_row, 0)

    @pl.when(s < NS - 1)
    def _():
        _zero_acc_range(acc, rows_v, s * NODE_BLK, NODE_BLK)

    @pl.when(s == NS - 1)
    def _():
        _zero_acc_range(acc, rows_v, (NS - 1) * NODE_BLK, NODE_BLK_LAST)

    plsc.subcore_barrier()

    rbase = s * RPT + jnp.minimum(s, REM)
    nrows = jnp.where(s < REM, RPT + 1, RPT)

    def scatter_edges(dest_hbm, attr_hbm):
        def fire_load(b, jj):
            pltpu.async_copy(dest_hbm.at[pl.ds((rbase + jj) * CHUNK, CHUNK)],
                             idx_r.at[b], lsems[b])
            pltpu.async_copy(attr_hbm.at[rbase + jj], rows_v.at[b], lsems[b])

        def wait_load(b, jj):
            pltpu.make_async_copy(dest_hbm.at[pl.ds((rbase + jj) * CHUNK,
                                                    CHUNK)],
                                  idx_r.at[b], lsems[b]).wait()
            pltpu.make_async_copy(attr_hbm.at[rbase + jj],
                                  rows_v.at[b], lsems[b]).wait()

        for b in range(NB - 1):
            fire_load(b, b)

        def chunk_body(j, _):
            def step(b):
                wait_load(b, j)
                pltpu.async_copy(rows_v.at[b], acc.at[idx_r.at[b]],
                                 ssems[b], add=True)
                bn = (b + NB - 1) % NB
                jn = j + NB - 1

                @pl.when(jn < nrows)
                def _():
                    @pl.when(j >= 1)
                    def _():
                        pltpu.make_async_copy(rows_v.at[bn],
                                              acc.at[idx_r.at[bn]],
                                              ssems[bn]).wait()
                    fire_load(bn, jn)

            for b in range(NB):
                @pl.when(j % NB == b)
                def _(b=b):
                    step(b)
            return 0

        lax.fori_loop(0, nrows, chunk_body, 0)

        for b in range(NB):
            pltpu.make_async_copy(rows_v.at[b], acc.at[idx_r.at[b]],
                                  ssems[b]).wait()

    @pl.when(c == 0)
    def _():
        scatter_edges(destw_hbm, attrw_hbm)

    @pl.when(c == 1)
    def _():
        scatter_edges(destm_hbm, attrm_hbm)

    plsc.subcore_barrier()

    def writeout(out_hbm):
        @pl.when(s < NS - 1)
        def _():
            pltpu.sync_copy(acc.at[pl.ds(s * NODE_BLK, NODE_BLK)],
                            out_hbm.at[pl.ds(s * NODE_BLK, NODE_BLK)])

        @pl.when(s == NS - 1)
        def _():
            pltpu.sync_copy(acc.at[pl.ds((NS - 1) * NODE_BLK, NODE_BLK_LAST)],
                            out_hbm.at[pl.ds((NS - 1) * NODE_BLK, NODE_BLK_LAST)])

    @pl.when(c == 0)
    def _():
        writeout(outw_hbm)

    @pl.when(c == 1)
    def _():
        writeout(outm_hbm)


def _segment_sums(destw, destm, attrw, attrm):
    mesh = plsc.VectorSubcoreMesh(core_axis_name="c", subcore_axis_name="s",
                                  num_cores=NC, num_subcores=NS)
    f = pl.kernel(
        _segsum_body,
        out_type=(jax.ShapeDtypeStruct((N_NODES, D), jnp.float32),
                  jax.ShapeDtypeStruct((N_NODES, D), jnp.float32)),
        mesh=mesh,
        scratch_types=[
            pltpu.VMEM_SHARED((N_NODES, D), jnp.float32),
            pltpu.VMEM((NB, CHUNK), jnp.int32),
            pltpu.VMEM((NB, CHUNK, D), jnp.float32),
        ] + [pltpu.SemaphoreType.DMA] * (2 * NB),
    )
    return f(destw, destm, attrw, attrm)


def _mlp_body(x_ref, ow_ref, om_ref, w0x_ref, w0w_ref, w0m_ref, b0_ref,
              w1_ref, b1_ref, w2_ref, b2_ref, out_ref):
    def silu(h):
        return h * (1.0 / (1.0 + jnp.exp(-h)))
    h = (jnp.dot(x_ref[...], w0x_ref[...], preferred_element_type=jnp.float32)
         + jnp.dot(ow_ref[...], w0w_ref[...], preferred_element_type=jnp.float32)
         + jnp.dot(om_ref[...], w0m_ref[...], preferred_element_type=jnp.float32)
         + b0_ref[...])
    h = silu(h)
    h = silu(jnp.dot(h, w1_ref[...], preferred_element_type=jnp.float32) + b1_ref[...])
    out_ref[...] = (jnp.dot(h, w2_ref[...], preferred_element_type=jnp.float32)
                    + b2_ref[...])


def _mlp(x, outw, outm, W0, b0, W1, b1, W2, b2):
    blk = 1000
    grid = (N_NODES // blk,)
    row_spec = pl.BlockSpec((blk, D), lambda i: (i, 0))
    w_spec = pl.BlockSpec((D, D), lambda i: (0, 0))
    b_spec = pl.BlockSpec((1, D), lambda i: (0, 0))
    return pl.pallas_call(
        _mlp_body,
        grid=grid,
        in_specs=[row_spec, row_spec, row_spec,
                  w_spec, w_spec, w_spec, b_spec,
                  w_spec, b_spec, w_spec, b_spec],
        out_specs=row_spec,
        out_shape=jax.ShapeDtypeStruct((N_NODES, D), jnp.float32),
    )(x, outw, outm, W0[0:D], W0[D:2 * D], W0[2 * D:3 * D], b0.reshape(1, D),
      W1, b1.reshape(1, D), W2, b2.reshape(1, D))


def kernel(x, edge_indexw, edge_indexm, edge_attrw, edge_attrm,
           W0, b0, W1, b1, W2, b2):
    destw = edge_indexw[1].astype(jnp.int32)
    destm = edge_indexm[1].astype(jnp.int32)
    attrw = edge_attrw.reshape(NCHUNKS, CHUNK, D)
    attrm = edge_attrm.reshape(NCHUNKS, CHUNK, D)
    outw, outm = _segment_sums(destw, destm, attrw, attrm)
    return _mlp(x, outw, outm, W0, b0, W1, b1, W2, b2)


# speedup vs baseline: 4.3889x; 1.0678x over previous
import jax
import jax.numpy as jnp
from jax import lax
from jax.experimental import pallas as pl
from jax.experimental.pallas import tpu as pltpu
from jax.experimental.pallas import tpu_sc as plsc

N_NODES = 10000
N_EDGES = 320000
D = 128

NC = 2
NS = 16

CHUNK = 64
NCHUNKS = N_EDGES // CHUNK
RPT = NCHUNKS // NS
REM = NCHUNKS - RPT * NS
NB = 6

NODE_BLK = 632
NODE_BLK_LAST = N_NODES - NODE_BLK * (NS - 1)
ZBLK = 64


def _zero_acc_range(acc, rows_v, base, count):
    nfull = count // ZBLK
    tail = count - nfull * ZBLK
    for k in range(nfull):
        pltpu.sync_copy(rows_v.at[0],
                        acc.at[pl.ds(base + k * ZBLK, ZBLK)])
    if tail:
        pltpu.sync_copy(rows_v.at[0, pl.ds(0, tail)],
                        acc.at[pl.ds(base + nfull * ZBLK, tail)])


def _segsum_body(destw_hbm, destm_hbm, attrw_hbm, attrm_hbm,
                 outw_hbm, outm_hbm, acc, idx_r, rows_v,
                 lsem0, lsem1, lsem2, lsem3, lsem4, lsem5,
                 ssem0, ssem1, ssem2, ssem3, ssem4, ssem5):
    c = lax.axis_index("c")
    s = lax.axis_index("s")
    lsems = (lsem0, lsem1, lsem2, lsem3, lsem4, lsem5)
    ssems = (ssem0, ssem1, ssem2, ssem3, ssem4, ssem5)

    def zero_row(r, _):
        for k in range(D // 16):
            rows_v[0, r, pl.ds(k * 16, 16)] = jnp.zeros((16,), jnp.float32)
        return 0
    lax.fori_loop(0, ZBLK, zero_row, 0)

    @pl.when(s < NS - 1)
    def _():
        _zero_acc_range(acc, rows_v, s * NODE_BLK, NODE_BLK)

    @pl.when(s == NS - 1)
    def _():
        _zero_acc_range(acc, rows_v, (NS - 1) * NODE_BLK, NODE_BLK_LAST)

    plsc.subcore_barrier()

    rbase = s * RPT + jnp.minimum(s, REM)
    nrows = jnp.where(s < REM, RPT + 1, RPT)

    def scatter_edges(dest_hbm, attr_hbm):
        def fire_load(b, jj):
            pltpu.async_copy(dest_hbm.at[pl.ds((rbase + jj) * CHUNK, CHUNK)],
                             idx_r.at[b], lsems[b])
            pltpu.async_copy(attr_hbm.at[rbase + jj], rows_v.at[b], lsems[b])

        def wait_load(b, jj):
            pltpu.make_async_copy(dest_hbm.at[pl.ds((rbase + jj) * CHUNK,
                                                    CHUNK)],
                                  idx_r.at[b], lsems[b]).wait()
            pltpu.make_async_copy(attr_hbm.at[rbase + jj],
                                  rows_v.at[b], lsems[b]).wait()

        for b in range(NB - 1):
            fire_load(b, b)

        def chunk_body(j, _):
            def step(b):
                wait_load(b, j)
                pltpu.async_copy(rows_v.at[b], acc.at[idx_r.at[b]],
                                 ssems[b], add=True)
                bn = (b + NB - 1) % NB
                jn = j + NB - 1

                @pl.when(jn < nrows)
                def _():
                    @pl.when(j >= 1)
                    def _():
                        pltpu.make_async_copy(rows_v.at[bn],
                                              acc.at[idx_r.at[bn]],
                                              ssems[bn]).wait()
                    fire_load(bn, jn)

            for b in range(NB):
                @pl.when(j % NB == b)
                def _(b=b):
                    step(b)
            return 0

        lax.fori_loop(0, nrows, chunk_body, 0)

        for b in range(NB):
            pltpu.make_async_copy(rows_v.at[b], acc.at[idx_r.at[b]],
                                  ssems[b]).wait()

    @pl.when(c == 0)
    def _():
        scatter_edges(destw_hbm, attrw_hbm)

    @pl.when(c == 1)
    def _():
        scatter_edges(destm_hbm, attrm_hbm)

    plsc.subcore_barrier()

    def writeout(out_hbm):
        @pl.when(s < NS - 1)
        def _():
            pltpu.sync_copy(acc.at[pl.ds(s * NODE_BLK, NODE_BLK)],
                            out_hbm.at[pl.ds(s * NODE_BLK, NODE_BLK)])

        @pl.when(s == NS - 1)
        def _():
            pltpu.sync_copy(acc.at[pl.ds((NS - 1) * NODE_BLK, NODE_BLK_LAST)],
                            out_hbm.at[pl.ds((NS - 1) * NODE_BLK, NODE_BLK_LAST)])

    @pl.when(c == 0)
    def _():
        writeout(outw_hbm)

    @pl.when(c == 1)
    def _():
        writeout(outm_hbm)


def _segment_sums(destw, destm, attrw, attrm):
    mesh = plsc.VectorSubcoreMesh(core_axis_name="c", subcore_axis_name="s",
                                  num_cores=NC, num_subcores=NS)
    f = pl.kernel(
        _segsum_body,
        out_type=(jax.ShapeDtypeStruct((N_NODES, D), jnp.float32),
                  jax.ShapeDtypeStruct((N_NODES, D), jnp.float32)),
        mesh=mesh,
        scratch_types=[
            pltpu.VMEM_SHARED((N_NODES, D), jnp.float32),
            pltpu.VMEM((NB, CHUNK), jnp.int32),
            pltpu.VMEM((NB, CHUNK, D), jnp.float32),
        ] + [pltpu.SemaphoreType.DMA] * (2 * NB),
    )
    return f(destw, destm, attrw, attrm)


def _mlp_body(x_ref, ow_ref, om_ref, w0x_ref, w0w_ref, w0m_ref, b0_ref,
              w1_ref, b1_ref, w2_ref, b2_ref, out_ref):
    def silu(h):
        return h * (1.0 / (1.0 + jnp.exp(-h)))
    h = (jnp.dot(x_ref[...], w0x_ref[...], preferred_element_type=jnp.float32)
         + jnp.dot(ow_ref[...], w0w_ref[...], preferred_element_type=jnp.float32)
         + jnp.dot(om_ref[...], w0m_ref[...], preferred_element_type=jnp.float32)
         + b0_ref[...])
    h = silu(h)
    h = silu(jnp.dot(h, w1_ref[...], preferred_element_type=jnp.float32) + b1_ref[...])
    out_ref[...] = (jnp.dot(h, w2_ref[...], preferred_element_type=jnp.float32)
                    + b2_ref[...])


def _mlp(x, outw, outm, W0, b0, W1, b1, W2, b2):
    blk = 1000
    grid = (N_NODES // blk,)
    row_spec = pl.BlockSpec((blk, D), lambda i: (i, 0))
    w_spec = pl.BlockSpec((D, D), lambda i: (0, 0))
    b_spec = pl.BlockSpec((1, D), lambda i: (0, 0))
    return pl.pallas_call(
        _mlp_body,
        grid=grid,
        in_specs=[row_spec, row_spec, row_spec,
                  w_spec, w_spec, w_spec, b_spec,
                  w_spec, b_spec, w_spec, b_spec],
        out_specs=row_spec,
        out_shape=jax.ShapeDtypeStruct((N_NODES, D), jnp.float32),
    )(x, outw, outm, W0[0:D], W0[D:2 * D], W0[2 * D:3 * D], b0.reshape(1, D),
      W1, b1.reshape(1, D), W2, b2.reshape(1, D))


def kernel(x, edge_indexw, edge_indexm, edge_attrw, edge_attrm,
           W0, b0, W1, b1, W2, b2):
    destw = edge_indexw[1].astype(jnp.int32)
    destm = edge_indexm[1].astype(jnp.int32)
    attrw = edge_attrw.reshape(NCHUNKS, CHUNK, D)
    attrm = edge_attrm.reshape(NCHUNKS, CHUNK, D)
    outw, outm = _segment_sums(destw, destm, attrw, attrm)
    return _mlp(x, outw, outm, W0, b0, W1, b1, W2, b2)


# speedup vs baseline: 4.6162x; 1.0518x over previous
import jax
import jax.numpy as jnp
from jax import lax
from jax.experimental import pallas as pl
from jax.experimental.pallas import tpu as pltpu
from jax.experimental.pallas import tpu_sc as plsc

N_NODES = 10000
N_EDGES = 320000
D = 128

NC = 2
NS = 16

CHUNK = 64
NCHUNKS = N_EDGES // CHUNK
RPT = NCHUNKS // NS
REM = NCHUNKS - RPT * NS
NB = 6

NODE_BLK = 632
NODE_BLK_LAST = N_NODES - NODE_BLK * (NS - 1)
ZBLK = 64


def _zero_acc_range(acc, rows_v, base, count):
    nfull = count // ZBLK
    tail = count - nfull * ZBLK
    for k in range(nfull):
        pltpu.sync_copy(rows_v.at[0],
                        acc.at[pl.ds(base + k * ZBLK, ZBLK)])
    if tail:
        pltpu.sync_copy(rows_v.at[0, pl.ds(0, tail)],
                        acc.at[pl.ds(base + nfull * ZBLK, tail)])


def _segsum_body(destw_hbm, destm_hbm, attrw_hbm, attrm_hbm,
                 outw_hbm, outm_hbm, acc, idx_r, rows_v,
                 lsem0, lsem1, lsem2, lsem3, lsem4, lsem5,
                 ssem0, ssem1, ssem2, ssem3, ssem4, ssem5):
    c = lax.axis_index("c")
    s = lax.axis_index("s")
    lsems = (lsem0, lsem1, lsem2, lsem3, lsem4, lsem5)
    ssems = (ssem0, ssem1, ssem2, ssem3, ssem4, ssem5)

    def zero_row(r, _):
        for k in range(D // 16):
            rows_v[0, r, pl.ds(k * 16, 16)] = jnp.zeros((16,), jnp.float32)
        return 0
    lax.fori_loop(0, ZBLK, zero_row, 0)

    @pl.when(s < NS - 1)
    def _():
        _zero_acc_range(acc, rows_v, s * NODE_BLK, NODE_BLK)

    @pl.when(s == NS - 1)
    def _():
        _zero_acc_range(acc, rows_v, (NS - 1) * NODE_BLK, NODE_BLK_LAST)

    plsc.subcore_barrier()

    rbase = s * RPT + jnp.minimum(s, REM)
    nrows = jnp.where(s < REM, RPT + 1, RPT)

    def scatter_edges(dest_hbm, attr_hbm):
        def fire_load(b, jj):
            pltpu.async_copy(
                dest_hbm.at[pl.ds(N_EDGES + (rbase + jj) * CHUNK, CHUNK)],
                idx_r.at[b], lsems[b])
            pltpu.async_copy(attr_hbm.at[rbase + jj], rows_v.at[b], lsems[b])

        def wait_load(b, jj):
            pltpu.make_async_copy(
                dest_hbm.at[pl.ds(N_EDGES + (rbase + jj) * CHUNK, CHUNK)],
                idx_r.at[b], lsems[b]).wait()
            pltpu.make_async_copy(attr_hbm.at[rbase + jj],
                                  rows_v.at[b], lsems[b]).wait()

        for b in range(NB - 1):
            fire_load(b, b)

        def chunk_body(j, _):
            def step(b):
                wait_load(b, j)
                pltpu.async_copy(rows_v.at[b], acc.at[idx_r.at[b]],
                                 ssems[b], add=True)
                bn = (b + NB - 1) % NB
                jn = j + NB - 1

                @pl.when(jn < nrows)
                def _():
                    @pl.when(j >= 1)
                    def _():
                        pltpu.make_async_copy(rows_v.at[bn],
                                              acc.at[idx_r.at[bn]],
                                              ssems[bn]).wait()
                    fire_load(bn, jn)

            for b in range(NB):
                @pl.when(j % NB == b)
                def _(b=b):
                    step(b)
            return 0

        lax.fori_loop(0, nrows, chunk_body, 0)

        for b in range(NB):
            pltpu.make_async_copy(rows_v.at[b], acc.at[idx_r.at[b]],
                                  ssems[b]).wait()

    @pl.when(c == 0)
    def _():
        scatter_edges(destw_hbm, attrw_hbm)

    @pl.when(c == 1)
    def _():
        scatter_edges(destm_hbm, attrm_hbm)

    plsc.subcore_barrier()

    def writeout(out_hbm):
        @pl.when(s < NS - 1)
        def _():
            pltpu.sync_copy(acc.at[pl.ds(s * NODE_BLK, NODE_BLK)],
                            out_hbm.at[pl.ds(s * NODE_BLK, NODE_BLK)])

        @pl.when(s == NS - 1)
        def _():
            pltpu.sync_copy(acc.at[pl.ds((NS - 1) * NODE_BLK, NODE_BLK_LAST)],
                            out_hbm.at[pl.ds((NS - 1) * NODE_BLK, NODE_BLK_LAST)])

    @pl.when(c == 0)
    def _():
        writeout(outw_hbm)

    @pl.when(c == 1)
    def _():
        writeout(outm_hbm)


def _segment_sums(destw, destm, attrw, attrm):
    mesh = plsc.VectorSubcoreMesh(core_axis_name="c", subcore_axis_name="s",
                                  num_cores=NC, num_subcores=NS)
    f = pl.kernel(
        _segsum_body,
        out_type=(jax.ShapeDtypeStruct((N_NODES, D), jnp.float32),
                  jax.ShapeDtypeStruct((N_NODES, D), jnp.float32)),
        mesh=mesh,
        scratch_types=[
            pltpu.VMEM_SHARED((N_NODES, D), jnp.float32),
            pltpu.VMEM((NB, CHUNK), jnp.int32),
            pltpu.VMEM((NB, CHUNK, D), jnp.float32),
        ] + [pltpu.SemaphoreType.DMA] * (2 * NB),
    )
    return f(destw, destm, attrw, attrm)


def _mlp_body(x_ref, ow_ref, om_ref, w0x_ref, w0w_ref, w0m_ref, b0_ref,
              w1_ref, b1_ref, w2_ref, b2_ref, out_ref):
    def silu(h):
        return h * (1.0 / (1.0 + jnp.exp(-h)))
    h = (jnp.dot(x_ref[...], w0x_ref[...], preferred_element_type=jnp.float32)
         + jnp.dot(ow_ref[...], w0w_ref[...], preferred_element_type=jnp.float32)
         + jnp.dot(om_ref[...], w0m_ref[...], preferred_element_type=jnp.float32)
         + b0_ref[...])
    h = silu(h)
    h = silu(jnp.dot(h, w1_ref[...], preferred_element_type=jnp.float32) + b1_ref[...])
    out_ref[...] = (jnp.dot(h, w2_ref[...], preferred_element_type=jnp.float32)
                    + b2_ref[...])


def _mlp(x, outw, outm, W0, b0, W1, b1, W2, b2):
    blk = 2000
    grid = (N_NODES // blk,)
    row_spec = pl.BlockSpec((blk, D), lambda i: (i, 0))
    w_spec = pl.BlockSpec((D, D), lambda i: (0, 0))
    b_spec = pl.BlockSpec((1, D), lambda i: (0, 0))
    return pl.pallas_call(
        _mlp_body,
        grid=grid,
        in_specs=[row_spec, row_spec, row_spec,
                  w_spec, w_spec, w_spec, b_spec,
                  w_spec, b_spec, w_spec, b_spec],
        out_specs=row_spec,
        out_shape=jax.ShapeDtypeStruct((N_NODES, D), jnp.float32),
    )(x, outw, outm, W0[0:D], W0[D:2 * D], W0[2 * D:3 * D], b0.reshape(1, D),
      W1, b1.reshape(1, D), W2, b2.reshape(1, D))


def kernel(x, edge_indexw, edge_indexm, edge_attrw, edge_attrm,
           W0, b0, W1, b1, W2, b2):
    destw = edge_indexw.astype(jnp.int32).reshape(2 * N_EDGES)
    destm = edge_indexm.astype(jnp.int32).reshape(2 * N_EDGES)
    attrw = edge_attrw.reshape(NCHUNKS, CHUNK, D)
    attrm = edge_attrm.reshape(NCHUNKS, CHUNK, D)
    outw, outm = _segment_sums(destw, destm, attrw, attrm)
    return _mlp(x, outw, outm, W0, b0, W1, b1, W2, b2)
